# Initial kernel scaffold; baseline (speedup 1.0000x reference)
import jax
import jax.numpy as jnp
from jax.experimental import pallas as pl


def kernel(imgs, mask_ratio, W_patch, b_patch, pos_embed):
    raise NotImplementedError("write your pallas kernel here")


# speedup vs baseline: 1.5089x; 1.5089x over previous
import jax
import jax.numpy as jnp
import numpy as np
from jax.experimental import pallas as pl
from jax.experimental.pallas import tpu as pltpu

_N = 64
_L = 196
_P = 16
_H = 14

_noise = jax.random.uniform(jax.random.key(1), (_N, _L), dtype=jnp.float32)
_ids_shuffle = jnp.argsort(_noise, axis=1)
_IDS_RESTORE = np.asarray(jnp.argsort(_ids_shuffle, axis=1)).reshape(_N, _H, _H)
del _noise, _ids_shuffle


def _mask_mul_kernel(mr_ref, ids_ref, img_ref, out_ref):
    len_keep = jnp.floor(_L * (1.0 - mr_ref[0]))
    keep = (ids_ref[0].astype(jnp.float32) < len_keep).astype(jnp.float32)

    r = jax.lax.broadcasted_iota(jnp.int32, (_P * _H, _H), 0) // _P
    c = jax.lax.broadcasted_iota(jnp.int32, (_P * _H, _H), 1)
    E = (r == c).astype(jnp.float32)
    rT = jax.lax.broadcasted_iota(jnp.int32, (_H, _P * _H), 0)
    cT = jax.lax.broadcasted_iota(jnp.int32, (_H, _P * _H), 1) // _P
    ET = (rT == cT).astype(jnp.float32)

    m = jnp.dot(E, jnp.dot(keep, ET, preferred_element_type=jnp.float32),
                preferred_element_type=jnp.float32)
    out_ref[0] = img_ref[0] * m[None, :, :]


def kernel(imgs, mask_ratio, W_patch, b_patch, pos_embed):
    del W_patch, b_patch, pos_embed
    ids = jnp.asarray(_IDS_RESTORE)
    mr = jnp.reshape(mask_ratio, (1,))
    return pl.pallas_call(
        _mask_mul_kernel,
        grid=(_N,),
        in_specs=[
            pl.BlockSpec(memory_space=pltpu.SMEM),
            pl.BlockSpec((1, _H, _H), lambda n: (n, 0, 0)),
            pl.BlockSpec((1, 3, 224, 224), lambda n: (n, 0, 0, 0)),
        ],
        out_specs=pl.BlockSpec((1, 3, 224, 224), lambda n: (n, 0, 0, 0)),
        out_shape=jax.ShapeDtypeStruct((_N, 3, 224, 224), jnp.float32),
    )(mr, ids, imgs)
